# Initial kernel scaffold; baseline (speedup 1.0000x reference)
import jax
import jax.numpy as jnp
from jax.experimental import pallas as pl


def kernel(x, pos, edge_index, w_rppe, b_rppe, w_att, w_post, b_post):
    raise NotImplementedError("write your pallas kernel here")


# speedup vs baseline: 1.0001x; 1.0001x over previous
import jax
import jax.numpy as jnp
from jax.experimental import pallas as pl


def kernel(x, pos, edge_index, w_rppe, b_rppe, w_att, w_post, b_post):
    src = edge_index[0]
    dst = edge_index[1]
    n = x.shape[0]

    pos_i = jnp.take(pos, dst, axis=0)
    pos_j = jnp.take(pos, src, axis=0)
    x_j = jnp.take(x, src, axis=0)

    delta = pos_i - pos_j
    dist = jnp.linalg.norm(delta, axis=-1, keepdims=True)
    rp = jnp.concatenate([pos_i, pos_j, delta, dist], axis=-1)
    r = jax.nn.relu(rp @ w_rppe + b_rppe)

    f_hat = jnp.concatenate([x_j, r], axis=-1)

    logits = f_hat @ w_att
    seg_max = jax.ops.segment_max(logits, dst, num_segments=n)
    seg_max = jnp.where(jnp.isfinite(seg_max), seg_max, 0.0)
    ex = jnp.exp(logits - jnp.take(seg_max, dst, axis=0))
    denom = jax.ops.segment_sum(ex, dst, num_segments=n)
    s_att = ex / (jnp.take(denom, dst, axis=0) + 1e-16)

    msg = s_att * f_hat
    agg = jax.ops.segment_sum(msg, dst, num_segments=n)

    out = jax.nn.relu(agg @ w_post + b_post)
    return out


# speedup vs baseline: 1.6881x; 1.6879x over previous
import functools

import jax
import jax.numpy as jnp
from jax.experimental import pallas as pl
from jax.experimental.pallas import tpu as pltpu

N = 100000
E = 1600000
D = 32

BE = 4000
BN = 4000


def _edge_payload_body(pi_ref, pj_ref, xj_ref, w1_ref, w2_ref, w9_ref,
                       watt_ref, out_ref):
    pi = pi_ref[...]
    pj = pj_ref[...]
    xj = xj_ref[...]
    delta = pi - pj
    dist = jnp.sqrt(jnp.sum(delta * delta, axis=1, keepdims=True))
    pre = (jnp.dot(pi, w1_ref[...], preferred_element_type=jnp.float32)
           + jnp.dot(pj, w2_ref[...], preferred_element_type=jnp.float32)
           + dist * w9_ref[0:1, :] + w9_ref[1:2, :])
    r = jnp.maximum(pre, 0.0)
    logits = (jnp.dot(xj, watt_ref[0:32, :], preferred_element_type=jnp.float32)
              + jnp.dot(r, watt_ref[32:64, :], preferred_element_type=jnp.float32))
    ex = jnp.exp(logits)
    out_ref[:, 0:32] = ex[:, 0:32] * xj
    out_ref[:, 32:64] = ex[:, 32:64] * r
    out_ref[:, 64:128] = ex


def _edge_payload(pi, pj, xj, w1, w2, w9b, w_att):
    grid = (E // BE,)
    return pl.pallas_call(
        _edge_payload_body,
        grid=grid,
        in_specs=[
            pl.BlockSpec((BE, 16), lambda i: (i, 0)),
            pl.BlockSpec((BE, 16), lambda i: (i, 0)),
            pl.BlockSpec((BE, 32), lambda i: (i, 0)),
            pl.BlockSpec((16, 32), lambda i: (0, 0)),
            pl.BlockSpec((16, 32), lambda i: (0, 0)),
            pl.BlockSpec((2, 32), lambda i: (0, 0)),
            pl.BlockSpec((64, 64), lambda i: (0, 0)),
        ],
        out_specs=pl.BlockSpec((BE, 128), lambda i: (i, 0)),
        out_shape=jax.ShapeDtypeStruct((E, 128), jnp.float32),
    )(pi, pj, xj, w1, w2, w9b, w_att)


def _final_body(acc_ref, wpost_ref, bpost_ref, out_ref):
    acc = acc_ref[...]
    agg = acc[:, 0:64] / (acc[:, 64:128] + 1e-20)
    out = jnp.dot(agg, wpost_ref[...], preferred_element_type=jnp.float32)
    out_ref[...] = jnp.maximum(out + bpost_ref[0:1, :], 0.0)


def _final(acc, w_post, b_post):
    grid = (N // BN,)
    return pl.pallas_call(
        _final_body,
        grid=grid,
        in_specs=[
            pl.BlockSpec((BN, 128), lambda i: (i, 0)),
            pl.BlockSpec((64, 32), lambda i: (0, 0)),
            pl.BlockSpec((1, 32), lambda i: (0, 0)),
        ],
        out_specs=pl.BlockSpec((BN, 32), lambda i: (i, 0)),
        out_shape=jax.ShapeDtypeStruct((N, 32), jnp.float32),
    )(acc, w_post, b_post.reshape(1, 32))


def kernel(x, pos, edge_index, w_rppe, b_rppe, w_att, w_post, b_post):
    src = edge_index[0]
    dst = edge_index[1]

    w1 = jnp.zeros((16, D), jnp.float32).at[0:3].set(w_rppe[0:3] + w_rppe[6:9])
    w2 = jnp.zeros((16, D), jnp.float32).at[0:3].set(w_rppe[3:6] - w_rppe[6:9])
    w9b = jnp.stack([w_rppe[9], b_rppe])

    pos16 = jnp.zeros((N, 16), jnp.float32).at[:, 0:3].set(pos)
    pi = jnp.take(pos16, dst, axis=0)
    pj = jnp.take(pos16, src, axis=0)
    xj = jnp.take(x, src, axis=0)

    payload = _edge_payload(pi, pj, xj, w1, w2, w9b, w_att)
    acc = jax.ops.segment_sum(payload, dst, num_segments=N)
    return _final(acc, w_post, b_post)


# speedup vs baseline: 4.5223x; 2.6789x over previous
import functools

import jax
import jax.numpy as jnp
from jax import lax
from jax.experimental import pallas as pl
from jax.experimental.pallas import tpu as pltpu
from jax.experimental.pallas import tpu_sc as plsc

N = 100000
E = 1600000
D = 32

BE = 4000
BN = 4000

NC = 2
NS = 16
NW = NC * NS
EPW = E // NW
GCH = 80
GSTEPS = EPW // GCH


def _sc_gather(table, src, dst):
    mesh = plsc.VectorSubcoreMesh(core_axis_name="c", subcore_axis_name="s")

    @functools.partial(
        pl.kernel,
        mesh=mesh,
        out_type=(
            jax.ShapeDtypeStruct((E, 128), jnp.float32),
            jax.ShapeDtypeStruct((E, 128), jnp.float32),
        ),
        scratch_types=[
            pltpu.VMEM((GCH,), jnp.int32),
            pltpu.VMEM((GCH,), jnp.int32),
            pltpu.VMEM((GCH, 128), jnp.float32),
            pltpu.VMEM((GCH, 128), jnp.float32),
            pltpu.SemaphoreType.DMA,
        ],
    )
    def gather_k(t_hbm, src_hbm, dst_hbm, rd_hbm, rs_hbm,
                 idxs_v, idxd_v, rs_v, rd_v, sem):
        wid = lax.axis_index("s") * NC + lax.axis_index("c")
        base = wid * EPW

        def step(i, carry):
            off = base + i * GCH
            pltpu.sync_copy(src_hbm.at[pl.ds(off, GCH)], idxs_v)
            pltpu.sync_copy(dst_hbm.at[pl.ds(off, GCH)], idxd_v)
            c1 = pltpu.async_copy(t_hbm.at[idxs_v], rs_v, sem)
            c2 = pltpu.async_copy(t_hbm.at[idxd_v], rd_v, sem)
            c1.wait()
            c2.wait()
            pltpu.sync_copy(rs_v, rs_hbm.at[pl.ds(off, GCH)])
            pltpu.sync_copy(rd_v, rd_hbm.at[pl.ds(off, GCH)])
            return carry

        lax.fori_loop(0, GSTEPS, step, 0)

    return gather_k(table, src, dst)


def _edge_payload_body(rd_ref, rs_ref, w1_ref, w2_ref, wx_ref, w9_ref,
                       watt2_ref, out_ref):
    rd = rd_ref[...]
    rs = rs_ref[...]
    lane = lax.broadcasted_iota(jnp.int32, (1, 128), 1)
    m = (lane < 16).astype(jnp.float32)
    d = (rd - rs) * m
    dist = jnp.sqrt(jnp.sum(d * d, axis=1, keepdims=True))
    pre = (jnp.dot(rd, w1_ref[...], preferred_element_type=jnp.float32)
           + jnp.dot(rs, w2_ref[...], preferred_element_type=jnp.float32)
           + dist * w9_ref[0:1, :] + w9_ref[1:2, :])
    r = jnp.maximum(pre, 0.0)
    logits = (jnp.dot(rs, wx_ref[...], preferred_element_type=jnp.float32)
              + jnp.dot(r, watt2_ref[...], preferred_element_type=jnp.float32))
    ex = jnp.exp(logits)
    xj = rs[:, 16:48]
    out_ref[:, 0:32] = ex[:, 0:32] * xj
    out_ref[:, 32:64] = ex[:, 32:64] * r
    out_ref[:, 64:128] = ex


def _edge_payload(rd, rs, w1, w2, wx, w9b, watt2):
    grid = (E // BE,)
    return pl.pallas_call(
        _edge_payload_body,
        grid=grid,
        in_specs=[
            pl.BlockSpec((BE, 128), lambda i: (i, 0)),
            pl.BlockSpec((BE, 128), lambda i: (i, 0)),
            pl.BlockSpec((128, 32), lambda i: (0, 0)),
            pl.BlockSpec((128, 32), lambda i: (0, 0)),
            pl.BlockSpec((128, 64), lambda i: (0, 0)),
            pl.BlockSpec((2, 32), lambda i: (0, 0)),
            pl.BlockSpec((32, 64), lambda i: (0, 0)),
        ],
        out_specs=pl.BlockSpec((BE, 128), lambda i: (i, 0)),
        out_shape=jax.ShapeDtypeStruct((E, 128), jnp.float32),
    )(rd, rs, w1, w2, wx, w9b, watt2)


def _final_body(acc_ref, wpost_ref, bpost_ref, out_ref):
    acc = acc_ref[...]
    agg = acc[:, 0:64] / (acc[:, 64:128] + 1e-20)
    out = jnp.dot(agg, wpost_ref[...], preferred_element_type=jnp.float32)
    out_ref[...] = jnp.maximum(out + bpost_ref[0:1, :], 0.0)


def _final(acc, w_post, b_post):
    grid = (N // BN,)
    return pl.pallas_call(
        _final_body,
        grid=grid,
        in_specs=[
            pl.BlockSpec((BN, 128), lambda i: (i, 0)),
            pl.BlockSpec((64, 32), lambda i: (0, 0)),
            pl.BlockSpec((1, 32), lambda i: (0, 0)),
        ],
        out_specs=pl.BlockSpec((BN, 32), lambda i: (i, 0)),
        out_shape=jax.ShapeDtypeStruct((N, 32), jnp.float32),
    )(acc, w_post, b_post.reshape(1, 32))


def kernel(x, pos, edge_index, w_rppe, b_rppe, w_att, w_post, b_post):
    src = edge_index[0]
    dst = edge_index[1]

    w1 = jnp.zeros((128, D), jnp.float32).at[0:3].set(w_rppe[0:3] + w_rppe[6:9])
    w2 = jnp.zeros((128, D), jnp.float32).at[0:3].set(w_rppe[3:6] - w_rppe[6:9])
    wx = jnp.zeros((128, 2 * D), jnp.float32).at[16:48].set(w_att[0:32])
    watt2 = w_att[32:64]
    w9b = jnp.stack([w_rppe[9], b_rppe])

    table = jnp.concatenate(
        [pos, jnp.zeros((N, 13), jnp.float32), x,
         jnp.zeros((N, 80), jnp.float32)], axis=1)
    rd, rs = _sc_gather(table, src, dst)

    payload = _edge_payload(rd, rs, w1, w2, wx, w9b, watt2)
    acc = jax.ops.segment_sum(payload, dst, num_segments=N)
    return _final(acc, w_post, b_post)
